# baseline (speedup 1.0000x reference)
import jax
import jax.numpy as jnp
import numpy as np
from jax.experimental import pallas as pl
from jax.experimental.pallas import tpu as pltpu

_RB = 448
_NRB = 7
_HW = 3136
_H = 56
_K = 1792
_NCOLS = 3136
_TILES = (640, 640, 640, 640, 576)
_Q2 = 224
_KC = 1024
_KX = 256 + _KC
_BIG = 3.0e38


def _build_kc():
    def resize_mat(n_in):
        sample_f = (np.arange(_H) + 0.5) * (n_in / _H) - 0.5
        x = np.abs(sample_f[:, None] - np.arange(n_in)[None, :])
        w = np.maximum(0.0, 1.0 - x)
        return (w / w.sum(axis=1, keepdims=True)).astype(np.float32)

    def pool_mat(n):
        idx = np.arange(n)
        return ((np.abs(idx[:, None] - idx[None, :]) <= 1) / 3.0).astype(
            np.float32)

    g1 = resize_mat(28) @ pool_mat(28)
    g2 = resize_mat(14) @ pool_mat(14)
    k1 = np.kron(g1, g1)
    k2 = np.kron(g2, g2)
    lin = np.linspace(-1.0, 1.0, _H, dtype=np.float32)
    coords = np.stack([np.tile(lin, _H), np.repeat(lin, _H)], axis=1)
    kc = np.concatenate(
        [k1, np.pad(k2, ((0, 0), (0, _Q2 - 196))), coords,
         np.zeros((_HW, _KC - 784 - _Q2 - 2), np.float32)], axis=1)
    import ml_dtypes
    return kc.astype(ml_dtypes.bfloat16)


_KC_CONST = _build_kc()


def _fused_kernel(t0_ref, kc_ref, p1_ref, p2_ref, cw_ref, b_ref, mb_ref,
                  out_ref, cent_ref, wcat_ref):
    b = pl.program_id(0)
    rb = pl.program_id(1)

    @pl.when(jnp.logical_and(b == 0, rb == 0))
    def _():
        off = 0
        for w in _TILES:
            sl = pl.ds(off, w)
            t = mb_ref[:, sl].astype(jnp.float32)
            cent_ref[:, sl] = jnp.sum(t * t, axis=0, keepdims=True)
            off += w
        wcat_ref[:, 0:256] = cw_ref[:, 0:256]
        wcat_ref[:, 1264:1280] = jnp.concatenate(
            [cw_ref[:, 1792:1794], jnp.zeros((_K, 14), jnp.bfloat16)],
            axis=1)

    @pl.when(rb == 0)
    def _():
        wcat_ref[:, 256:1040] = jnp.dot(
            cw_ref[:, 256:768], p1_ref[0],
            preferred_element_type=jnp.float32).astype(jnp.bfloat16)
        wcat_ref[:, 1040:1264] = jnp.dot(
            cw_ref[:, 768:1792], p2_ref[0],
            preferred_element_type=jnp.float32).astype(jnp.bfloat16)

    x = jnp.concatenate([t0_ref[0], kc_ref[...]], axis=1)
    phi = jax.lax.dot_general(
        x, wcat_ref[...], (((1,), (1,)), ((), ())),
        preferred_element_type=jnp.float32) + b_ref[...]

    feat = jnp.sum(phi * phi, axis=1, keepdims=True)
    phib = (2.0 * phi).astype(jnp.bfloat16)

    r0 = jnp.full((_RB, 128), _BIG, jnp.float32)
    r1 = r0
    r2 = r0
    off = 0
    for w in _TILES:
        sl = pl.ds(off, w)
        d = cent_ref[:, sl] - jnp.dot(
            phib, mb_ref[:, sl], preferred_element_type=jnp.float32)
        off += w
        for s in range(0, w, 128):
            ds_ = d[:, s:s + 128]
            if ds_.shape[1] < 128:
                ds_ = jnp.concatenate(
                    [ds_, jnp.full((_RB, 128 - ds_.shape[1]), _BIG,
                                   jnp.float32)], axis=1)
            hi0 = jnp.maximum(r0, ds_)
            r0 = jnp.minimum(r0, ds_)
            hi1 = jnp.maximum(r1, hi0)
            r1 = jnp.minimum(r1, hi0)
            r2 = jnp.minimum(r2, hi1)

    iota = jax.lax.broadcasted_iota(jnp.int32, (_RB, 128), 1)
    ds = []
    for _ in range(3):
        dmin = jnp.min(r0, axis=1, keepdims=True)
        sel = jnp.where(r0 == dmin, iota, jnp.int32(2 ** 30))
        li = jnp.min(sel, axis=1, keepdims=True)
        m = iota == li
        r0 = jnp.where(m, r1, r0)
        r1 = jnp.where(m, r2, r1)
        r2 = jnp.where(m, _BIG, r2)
        ds.append(dmin)

    d0, d1, d2 = [jnp.sqrt(jnp.maximum(feat + x_, 1e-12)) for x_ in ds]
    score = d0 / (1.0 + jnp.exp(d0 - d1) + jnp.exp(d0 - d2))
    out_ref[...] = score[None]


@jax.jit
def kernel(p0, p1, p2, conv_w, conv_b, memory_bank):
    B = p0.shape[0]
    f32, bf16 = jnp.float32, jnp.bfloat16

    q0 = p0.transpose(0, 2, 3, 1)
    t0 = jax.lax.reduce_window(q0, 0.0, jax.lax.add, (1, 3, 3, 1),
                               (1, 1, 1, 1), 'SAME') / 9.0
    t0 = t0.reshape(B, _HW, 256).astype(bf16)

    p1v = p1.reshape(B, 512, 784).astype(bf16)
    p2v = jnp.pad(p2.reshape(B, 1024, 196),
                  ((0, 0), (0, 0), (0, _Q2 - 196))).astype(bf16)

    kc = jnp.asarray(_KC_CONST)
    cw = conv_w.astype(bf16)
    b_row = conv_b.reshape(1, _K).astype(f32)

    mb = memory_bank.astype(bf16)

    grid = (B, _NRB)
    score = pl.pallas_call(
        _fused_kernel,
        grid=grid,
        in_specs=[
            pl.BlockSpec((1, _RB, 256), lambda b, r: (b, r, 0)),
            pl.BlockSpec((_RB, _KC), lambda b, r: (r, 0)),
            pl.BlockSpec((1, 512, 784), lambda b, r: (b, 0, 0)),
            pl.BlockSpec((1, 1024, _Q2), lambda b, r: (b, 0, 0)),
            pl.BlockSpec((_K, 1794), lambda b, r: (0, 0)),
            pl.BlockSpec((1, _K), lambda b, r: (0, 0)),
            pl.BlockSpec((_K, _NCOLS), lambda b, r: (0, 0)),
        ],
        out_specs=pl.BlockSpec((1, _RB, 1), lambda b, r: (b, r, 0)),
        out_shape=jax.ShapeDtypeStruct((B, _HW, 1), f32),
        scratch_shapes=[
            pltpu.VMEM((1, _NCOLS), f32),
            pltpu.VMEM((_K, _KX), jnp.bfloat16),
        ],
    )(t0, kc, p1v, p2v, cw, b_row, mb)

    score = score.reshape(B, _H, _H)[:, None, :, :]
    return (jnp.zeros(()), score)


# speedup vs baseline: 1.0140x; 1.0140x over previous
import jax
import jax.numpy as jnp
import numpy as np
from jax.experimental import pallas as pl
from jax.experimental.pallas import tpu as pltpu

_RB = 448
_NRB = 7
_HW = 3136
_H = 56
_K = 1792
_NCOLS = 3136
_TILES = (640, 640, 640, 640, 576)
_Q2 = 224
_KC = 1024
_KX = 256 + _KC
_BIG = 3.0e38


def _build_kc():
    def resize_mat(n_in):
        sample_f = (np.arange(_H) + 0.5) * (n_in / _H) - 0.5
        x = np.abs(sample_f[:, None] - np.arange(n_in)[None, :])
        w = np.maximum(0.0, 1.0 - x)
        return (w / w.sum(axis=1, keepdims=True)).astype(np.float32)

    def pool_mat(n):
        idx = np.arange(n)
        return ((np.abs(idx[:, None] - idx[None, :]) <= 1) / 3.0).astype(
            np.float32)

    g1 = resize_mat(28) @ pool_mat(28)
    g2 = resize_mat(14) @ pool_mat(14)
    k1 = np.kron(g1, g1)
    k2 = np.kron(g2, g2)
    lin = np.linspace(-1.0, 1.0, _H, dtype=np.float32)
    coords = np.stack([np.tile(lin, _H), np.repeat(lin, _H)], axis=1)
    kc = np.concatenate(
        [k1, np.pad(k2, ((0, 0), (0, _Q2 - 196))), coords,
         np.zeros((_HW, _KC - 784 - _Q2 - 2), np.float32)], axis=1)
    import ml_dtypes
    return kc.astype(ml_dtypes.bfloat16)


_KC_CONST = _build_kc()


def _fused_kernel(nsteps, t0_ref, kc_ref, q1_ref, q2_ref,
                  w0_ref, w1_ref, w2_ref, wxy_ref, b_ref, mb_ref,
                  out_ref, cent_ref, wcat_ref, r0_ref, r1_ref, r2_ref,
                  feat_ref):
    s = pl.program_id(0)

    @pl.when(s > 0)
    def _():
        r0 = r0_ref[...]
        r1 = r1_ref[...]
        r2 = r2_ref[...]
        feat = feat_ref[...]
        iota = jax.lax.broadcasted_iota(jnp.int32, (_RB, 128), 1)
        ds = []
        for _ in range(3):
            dmin = jnp.min(r0, axis=1, keepdims=True)
            sel = jnp.where(r0 == dmin, iota, jnp.int32(2 ** 30))
            li = jnp.min(sel, axis=1, keepdims=True)
            m = iota == li
            r0 = jnp.where(m, r1, r0)
            r1 = jnp.where(m, r2, r1)
            r2 = jnp.where(m, _BIG, r2)
            ds.append(dmin)
        d0, d1, d2 = [jnp.sqrt(jnp.maximum(feat + x_, 1e-12)) for x_ in ds]
        score = d0 / (1.0 + jnp.exp(d0 - d1) + jnp.exp(d0 - d2))
        out_ref[...] = score[None]

    @pl.when(s == 0)
    def _():
        off = 0
        for w in _TILES:
            sl = pl.ds(off, w)
            t = mb_ref[:, sl].astype(jnp.float32)
            cent_ref[:, sl] = jnp.sum(t * t, axis=0, keepdims=True)
            off += w
        wcat_ref[0:256, :] = w0_ref[...]
        wcat_ref[1264:1280, :] = jnp.concatenate(
            [wxy_ref[...], jnp.zeros((14, _K), jnp.bfloat16)], axis=0)

    @pl.when(jnp.logical_and(s % _NRB == 0, s < nsteps))
    def _():
        wcat_ref[256:1040, :] = jnp.dot(
            q1_ref[0], w1_ref[...],
            preferred_element_type=jnp.float32).astype(jnp.bfloat16)
        wcat_ref[1040:1264, :] = jnp.dot(
            q2_ref[0], w2_ref[...],
            preferred_element_type=jnp.float32).astype(jnp.bfloat16)

    @pl.when(s < nsteps)
    def _():
        x = jnp.concatenate([t0_ref[0], kc_ref[...]], axis=1)
        phi = jnp.dot(x, wcat_ref[...],
                      preferred_element_type=jnp.float32) + b_ref[...]

        feat_ref[...] = jnp.sum(phi * phi, axis=1, keepdims=True)
        phib = (2.0 * phi).astype(jnp.bfloat16)

        r0 = jnp.full((_RB, 128), _BIG, jnp.float32)
        r1 = r0
        r2 = r0
        off = 0
        for w in _TILES:
            sl = pl.ds(off, w)
            d = cent_ref[:, sl] - jnp.dot(
                phib, mb_ref[:, sl], preferred_element_type=jnp.float32)
            off += w
            for c in range(0, w, 128):
                ds_ = d[:, c:c + 128]
                if ds_.shape[1] < 128:
                    ds_ = jnp.concatenate(
                        [ds_, jnp.full((_RB, 128 - ds_.shape[1]), _BIG,
                                       jnp.float32)], axis=1)
                hi0 = jnp.maximum(r0, ds_)
                r0 = jnp.minimum(r0, ds_)
                hi1 = jnp.maximum(r1, hi0)
                r1 = jnp.minimum(r1, hi0)
                r2 = jnp.minimum(r2, hi1)
        r0_ref[...] = r0
        r1_ref[...] = r1
        r2_ref[...] = r2


@jax.jit
def kernel(p0, p1, p2, conv_w, conv_b, memory_bank):
    B = p0.shape[0]
    f32, bf16 = jnp.float32, jnp.bfloat16

    q0 = p0.transpose(0, 2, 3, 1)
    t0 = jax.lax.reduce_window(q0, 0.0, jax.lax.add, (1, 3, 3, 1),
                               (1, 1, 1, 1), 'SAME') / 9.0
    t0 = t0.reshape(B, _HW, 256).astype(bf16)

    q1 = p1.transpose(0, 2, 3, 1).reshape(B, 784, 512).astype(bf16)
    q2 = p2.transpose(0, 2, 3, 1).reshape(B, 196, 1024)
    q2 = jnp.pad(q2, ((0, 0), (0, _Q2 - 196), (0, 0))).astype(bf16)

    kc = jnp.asarray(_KC_CONST)

    wt = conv_w.T
    w0 = wt[0:256].astype(bf16)
    w1 = wt[256:768].astype(bf16)
    w2 = wt[768:1792].astype(bf16)
    wxy = wt[1792:1794].astype(bf16)
    b_row = conv_b.reshape(1, _K).astype(f32)

    mb = memory_bank.astype(bf16)

    nsteps = B * _NRB
    bmax = B - 1
    import functools
    body = functools.partial(_fused_kernel, nsteps)
    score = pl.pallas_call(
        body,
        grid=(nsteps + 1,),
        in_specs=[
            pl.BlockSpec((1, _RB, 256),
                         lambda s: (jnp.minimum(s // _NRB, bmax),
                                    s % _NRB, 0)),
            pl.BlockSpec((_RB, _KC), lambda s: (s % _NRB, 0)),
            pl.BlockSpec((1, 784, 512),
                         lambda s: (jnp.minimum(s // _NRB, bmax),
                                    0, 0)),
            pl.BlockSpec((1, _Q2, 1024),
                         lambda s: (jnp.minimum(s // _NRB, bmax),
                                    0, 0)),
            pl.BlockSpec((256, _K), lambda s: (0, 0)),
            pl.BlockSpec((512, _K), lambda s: (0, 0)),
            pl.BlockSpec((1024, _K), lambda s: (0, 0)),
            pl.BlockSpec((2, _K), lambda s: (0, 0)),
            pl.BlockSpec((1, _K), lambda s: (0, 0)),
            pl.BlockSpec((_K, _NCOLS), lambda s: (0, 0)),
        ],
        out_specs=pl.BlockSpec(
            (1, _RB, 1),
            lambda s: ((jnp.maximum(s - 1, 0)) // _NRB,
                       (jnp.maximum(s - 1, 0)) % _NRB, 0)),
        out_shape=jax.ShapeDtypeStruct((B, _HW, 1), f32),
        scratch_shapes=[
            pltpu.VMEM((1, _NCOLS), f32),
            pltpu.VMEM((_KX, _K), jnp.bfloat16),
            pltpu.VMEM((_RB, 128), f32),
            pltpu.VMEM((_RB, 128), f32),
            pltpu.VMEM((_RB, 128), f32),
            pltpu.VMEM((_RB, 1), f32),
        ],
    )(t0, kc, q1, q2, w0, w1, w2, wxy, b_row, mb)

    score = score.reshape(B, _H, _H)[:, None, :, :]
    return (jnp.zeros(()), score)


# speedup vs baseline: 1.0821x; 1.0672x over previous
import jax
import jax.numpy as jnp
import numpy as np
from jax.experimental import pallas as pl
from jax.experimental.pallas import tpu as pltpu

_RB = 448
_NRB = 7
_HW = 3136
_H = 56
_K = 1792
_NCOLS = 3136
_TILES = (512, 512, 512, 512, 512, 512, 64)
_Q2 = 224
_KC = 1024
_KX = 256 + _KC
_BIG = 3.0e38


def _build_kc():
    def resize_mat(n_in):
        sample_f = (np.arange(_H) + 0.5) * (n_in / _H) - 0.5
        x = np.abs(sample_f[:, None] - np.arange(n_in)[None, :])
        w = np.maximum(0.0, 1.0 - x)
        return (w / w.sum(axis=1, keepdims=True)).astype(np.float32)

    def pool_mat(n):
        idx = np.arange(n)
        return ((np.abs(idx[:, None] - idx[None, :]) <= 1) / 3.0).astype(
            np.float32)

    g1 = resize_mat(28) @ pool_mat(28)
    g2 = resize_mat(14) @ pool_mat(14)
    k1 = np.kron(g1, g1)
    k2 = np.kron(g2, g2)
    lin = np.linspace(-1.0, 1.0, _H, dtype=np.float32)
    coords = np.stack([np.tile(lin, _H), np.repeat(lin, _H)], axis=1)
    kc = np.concatenate(
        [k1, np.pad(k2, ((0, 0), (0, _Q2 - 196))), coords,
         np.zeros((_HW, _KC - 784 - _Q2 - 2), np.float32)], axis=1)
    import ml_dtypes
    return kc.astype(ml_dtypes.bfloat16)


_KC_CONST = _build_kc()


def _fused_kernel(t0_ref, kc_ref, q1_ref, q2_ref,
                  w0_ref, w1_ref, w2_ref, wxy_ref, b_ref, mb_ref,
                  out_ref, cent_ref, wcat_ref):
    b = pl.program_id(0)
    rb = pl.program_id(1)

    @pl.when(jnp.logical_and(b == 0, rb == 0))
    def _():
        off = 0
        for w in _TILES:
            sl = pl.ds(off, w)
            t = mb_ref[:, sl].astype(jnp.float32)
            cent_ref[:, sl] = jnp.sum(t * t, axis=0, keepdims=True)
            off += w
        wcat_ref[0:256, :] = w0_ref[...]
        wcat_ref[1264:1280, :] = jnp.concatenate(
            [wxy_ref[...], jnp.zeros((14, _K), jnp.bfloat16)], axis=0)

    @pl.when(rb == 0)
    def _():
        wcat_ref[256:1040, :] = jnp.dot(
            q1_ref[0], w1_ref[...],
            preferred_element_type=jnp.float32).astype(jnp.bfloat16)
        wcat_ref[1040:1264, :] = jnp.dot(
            q2_ref[0], w2_ref[...],
            preferred_element_type=jnp.float32).astype(jnp.bfloat16)

    x = jnp.concatenate([t0_ref[0], kc_ref[...]], axis=1)
    phi = jnp.dot(x, wcat_ref[...],
                  preferred_element_type=jnp.float32) + b_ref[...]

    feat = jnp.sum(phi * phi, axis=1, keepdims=True)
    phib = (2.0 * phi).astype(jnp.bfloat16)

    r0 = jnp.full((_RB, 128), _BIG, jnp.float32)
    r1 = r0
    r2 = r0
    off = 0
    for w in _TILES:
        sl = pl.ds(off, w)
        d = cent_ref[:, sl] - jnp.dot(
            phib, mb_ref[:, sl], preferred_element_type=jnp.float32)
        off += w
        for s in range(0, w, 128):
            ds_ = d[:, s:s + 128]
            if ds_.shape[1] < 128:
                ds_ = jnp.concatenate(
                    [ds_, jnp.full((_RB, 128 - ds_.shape[1]), _BIG,
                                   jnp.float32)], axis=1)
            hi0 = jnp.maximum(r0, ds_)
            r0 = jnp.minimum(r0, ds_)
            hi1 = jnp.maximum(r1, hi0)
            r1 = jnp.minimum(r1, hi0)
            r2 = jnp.minimum(r2, hi1)

    iota = jax.lax.broadcasted_iota(jnp.int32, (_RB, 128), 1)
    ds = []
    for _ in range(3):
        dmin = jnp.min(r0, axis=1, keepdims=True)
        sel = jnp.where(r0 == dmin, iota, jnp.int32(2 ** 30))
        li = jnp.min(sel, axis=1, keepdims=True)
        m = iota == li
        r0 = jnp.where(m, r1, r0)
        r1 = jnp.where(m, r2, r1)
        r2 = jnp.where(m, _BIG, r2)
        ds.append(dmin)

    d0, d1, d2 = [jnp.sqrt(jnp.maximum(feat + x_, 1e-12)) for x_ in ds]
    score = d0 / (1.0 + jnp.exp(d0 - d1) + jnp.exp(d0 - d2))
    out_ref[...] = score[None]


@jax.jit
def kernel(p0, p1, p2, conv_w, conv_b, memory_bank):
    B = p0.shape[0]
    f32, bf16 = jnp.float32, jnp.bfloat16

    q0 = p0.transpose(0, 2, 3, 1)
    t0 = jax.lax.reduce_window(q0, 0.0, jax.lax.add, (1, 3, 3, 1),
                               (1, 1, 1, 1), 'SAME') / 9.0
    t0 = t0.reshape(B, _HW, 256).astype(bf16)

    q1 = p1.transpose(0, 2, 3, 1).reshape(B, 784, 512).astype(bf16)
    q2 = p2.transpose(0, 2, 3, 1).reshape(B, 196, 1024)
    q2 = jnp.pad(q2, ((0, 0), (0, _Q2 - 196), (0, 0))).astype(bf16)

    kc = jnp.asarray(_KC_CONST)

    wt = conv_w.T
    w0 = wt[0:256].astype(bf16)
    w1 = wt[256:768].astype(bf16)
    w2 = wt[768:1792].astype(bf16)
    wxy = wt[1792:1794].astype(bf16)
    b_row = conv_b.reshape(1, _K).astype(f32)

    mb = memory_bank.astype(bf16)

    grid = (B, _NRB)
    score = pl.pallas_call(
        _fused_kernel,
        grid=grid,
        in_specs=[
            pl.BlockSpec((1, _RB, 256), lambda b, r: (b, r, 0)),
            pl.BlockSpec((_RB, _KC), lambda b, r: (r, 0)),
            pl.BlockSpec((1, 784, 512), lambda b, r: (b, 0, 0)),
            pl.BlockSpec((1, _Q2, 1024), lambda b, r: (b, 0, 0)),
            pl.BlockSpec((256, _K), lambda b, r: (0, 0)),
            pl.BlockSpec((512, _K), lambda b, r: (0, 0)),
            pl.BlockSpec((1024, _K), lambda b, r: (0, 0)),
            pl.BlockSpec((2, _K), lambda b, r: (0, 0)),
            pl.BlockSpec((1, _K), lambda b, r: (0, 0)),
            pl.BlockSpec((_K, _NCOLS), lambda b, r: (0, 0)),
        ],
        out_specs=pl.BlockSpec((1, _RB, 1), lambda b, r: (b, r, 0)),
        out_shape=jax.ShapeDtypeStruct((B, _HW, 1), f32),
        scratch_shapes=[
            pltpu.VMEM((1, _NCOLS), f32),
            pltpu.VMEM((_KX, _K), jnp.bfloat16),
        ],
    )(t0, kc, q1, q2, w0, w1, w2, wxy, b_row, mb)

    score = score.reshape(B, _H, _H)[:, None, :, :]
    return (jnp.zeros(()), score)


# speedup vs baseline: 1.1111x; 1.0268x over previous
import jax
import jax.numpy as jnp
import numpy as np
from jax.experimental import pallas as pl
from jax.experimental.pallas import tpu as pltpu

_RB = 784
_NRB = 4
_HW = 3136
_H = 56
_K = 1792
_NCOLS = 3136
_TILES = (512, 512, 512, 512, 512, 512, 64)
_Q2 = 224
_KC = 1024
_KX = 256 + _KC
_BIG = 3.0e38


def _build_kc():
    def resize_mat(n_in):
        sample_f = (np.arange(_H) + 0.5) * (n_in / _H) - 0.5
        x = np.abs(sample_f[:, None] - np.arange(n_in)[None, :])
        w = np.maximum(0.0, 1.0 - x)
        return (w / w.sum(axis=1, keepdims=True)).astype(np.float32)

    def pool_mat(n):
        idx = np.arange(n)
        return ((np.abs(idx[:, None] - idx[None, :]) <= 1) / 3.0).astype(
            np.float32)

    g1 = resize_mat(28) @ pool_mat(28)
    g2 = resize_mat(14) @ pool_mat(14)
    k1 = np.kron(g1, g1)
    k2 = np.kron(g2, g2)
    lin = np.linspace(-1.0, 1.0, _H, dtype=np.float32)
    coords = np.stack([np.tile(lin, _H), np.repeat(lin, _H)], axis=1)
    kc = np.concatenate(
        [k1, np.pad(k2, ((0, 0), (0, _Q2 - 196))), coords,
         np.zeros((_HW, _KC - 784 - _Q2 - 2), np.float32)], axis=1)
    import ml_dtypes
    return kc.astype(ml_dtypes.bfloat16)


_KC_CONST = _build_kc()


def _fused_kernel(t0_ref, kc_ref, q1_ref, q2_ref,
                  w0_ref, w1_ref, w2_ref, wxy_ref, b_ref, mb_ref,
                  out_ref, cent_ref, wcat_ref):
    b = pl.program_id(0)
    rb = pl.program_id(1)

    @pl.when(jnp.logical_and(b == 0, rb == 0))
    def _():
        off = 0
        for w in _TILES:
            sl = pl.ds(off, w)
            t = mb_ref[:, sl].astype(jnp.float32)
            cent_ref[:, sl] = jnp.sum(t * t, axis=0, keepdims=True)
            off += w
        wcat_ref[0:256, :] = w0_ref[...]
        wcat_ref[1264:1280, :] = jnp.concatenate(
            [wxy_ref[...], jnp.zeros((14, _K), jnp.bfloat16)], axis=0)

    @pl.when(rb == 0)
    def _():
        wcat_ref[256:1040, :] = jnp.dot(
            q1_ref[0], w1_ref[...],
            preferred_element_type=jnp.float32).astype(jnp.bfloat16)
        wcat_ref[1040:1264, :] = jnp.dot(
            q2_ref[0], w2_ref[...],
            preferred_element_type=jnp.float32).astype(jnp.bfloat16)

    x = jnp.concatenate([t0_ref[0], kc_ref[...]], axis=1)
    phi = jnp.dot(x, wcat_ref[...],
                  preferred_element_type=jnp.float32) + b_ref[...]

    feat = jnp.sum(phi * phi, axis=1, keepdims=True)
    phib = (2.0 * phi).astype(jnp.bfloat16)

    r0 = jnp.full((_RB, 128), _BIG, jnp.float32)
    r1 = r0
    r2 = r0
    off = 0
    for w in _TILES:
        sl = pl.ds(off, w)
        d = cent_ref[:, sl] - jnp.dot(
            phib, mb_ref[:, sl], preferred_element_type=jnp.float32)
        off += w
        for s in range(0, w, 128):
            ds_ = d[:, s:s + 128]
            if ds_.shape[1] < 128:
                ds_ = jnp.concatenate(
                    [ds_, jnp.full((_RB, 128 - ds_.shape[1]), _BIG,
                                   jnp.float32)], axis=1)
            hi0 = jnp.maximum(r0, ds_)
            r0 = jnp.minimum(r0, ds_)
            hi1 = jnp.maximum(r1, hi0)
            r1 = jnp.minimum(r1, hi0)
            r2 = jnp.minimum(r2, hi1)

    iota = jax.lax.broadcasted_iota(jnp.int32, (_RB, 128), 1)
    ds = []
    for _ in range(3):
        dmin = jnp.min(r0, axis=1, keepdims=True)
        sel = jnp.where(r0 == dmin, iota, jnp.int32(2 ** 30))
        li = jnp.min(sel, axis=1, keepdims=True)
        m = iota == li
        r0 = jnp.where(m, r1, r0)
        r1 = jnp.where(m, r2, r1)
        r2 = jnp.where(m, _BIG, r2)
        ds.append(dmin)

    d0, d1, d2 = [jnp.sqrt(jnp.maximum(feat + x_, 1e-12)) for x_ in ds]
    score = d0 / (1.0 + jnp.exp(d0 - d1) + jnp.exp(d0 - d2))
    out_ref[...] = score[None]


@jax.jit
def kernel(p0, p1, p2, conv_w, conv_b, memory_bank):
    B = p0.shape[0]
    f32, bf16 = jnp.float32, jnp.bfloat16

    q0 = p0.transpose(0, 2, 3, 1)
    t0 = jax.lax.reduce_window(q0, 0.0, jax.lax.add, (1, 3, 3, 1),
                               (1, 1, 1, 1), 'SAME') / 9.0
    t0 = t0.reshape(B, _HW, 256).astype(bf16)

    q1 = p1.transpose(0, 2, 3, 1).reshape(B, 784, 512).astype(bf16)
    q2 = p2.transpose(0, 2, 3, 1).reshape(B, 196, 1024)
    q2 = jnp.pad(q2, ((0, 0), (0, _Q2 - 196), (0, 0))).astype(bf16)

    kc = jnp.asarray(_KC_CONST)

    wt = conv_w.T
    w0 = wt[0:256].astype(bf16)
    w1 = wt[256:768].astype(bf16)
    w2 = wt[768:1792].astype(bf16)
    wxy = wt[1792:1794].astype(bf16)
    b_row = conv_b.reshape(1, _K).astype(f32)

    mb = memory_bank.astype(bf16)

    grid = (B, _NRB)
    score = pl.pallas_call(
        _fused_kernel,
        grid=grid,
        in_specs=[
            pl.BlockSpec((1, _RB, 256), lambda b, r: (b, r, 0)),
            pl.BlockSpec((_RB, _KC), lambda b, r: (r, 0)),
            pl.BlockSpec((1, 784, 512), lambda b, r: (b, 0, 0)),
            pl.BlockSpec((1, _Q2, 1024), lambda b, r: (b, 0, 0)),
            pl.BlockSpec((256, _K), lambda b, r: (0, 0)),
            pl.BlockSpec((512, _K), lambda b, r: (0, 0)),
            pl.BlockSpec((1024, _K), lambda b, r: (0, 0)),
            pl.BlockSpec((2, _K), lambda b, r: (0, 0)),
            pl.BlockSpec((1, _K), lambda b, r: (0, 0)),
            pl.BlockSpec((_K, _NCOLS), lambda b, r: (0, 0)),
        ],
        out_specs=pl.BlockSpec((1, _RB, 1), lambda b, r: (b, r, 0)),
        out_shape=jax.ShapeDtypeStruct((B, _HW, 1), f32),
        scratch_shapes=[
            pltpu.VMEM((1, _NCOLS), f32),
            pltpu.VMEM((_KX, _K), jnp.bfloat16),
        ],
    )(t0, kc, q1, q2, w0, w1, w2, wxy, b_row, mb)

    score = score.reshape(B, _H, _H)[:, None, :, :]
    return (jnp.zeros(()), score)
